# baseline (speedup 1.0000x reference)
import functools

import jax
import jax.numpy as jnp
from jax import lax
from jax.experimental import pallas as pl
from jax.experimental.pallas import tpu as pltpu
from jax.experimental.pallas import tpu_sc as plsc

B = 16384
H = 20
D = 64
NUM_OUT = 3
V = 1000000
VH = V // 2

NC = 2
NS = 16
NW = NC * NS
BPW = B // NW
BPC = 4
IPC = BPC * H
NCH = BPW // BPC
IPW = BPW * H
DV = D // 16


def _repack_body(a_ref, b_ref, out_ref):
    out_ref[:, 0:D] = a_ref[...]
    out_ref[:, D:2 * D] = b_ref[...]


def _repack(table):
    blkr = 4000
    nb = VH // blkr
    return pl.pallas_call(
        _repack_body,
        grid=(nb,),
        in_specs=[pl.BlockSpec((blkr, D), lambda i: (i, 0)),
                  pl.BlockSpec((blkr, D), lambda i: (i + nb, 0))],
        out_specs=pl.BlockSpec((blkr, 2 * D), lambda i: (i, 0)),
        out_shape=jax.ShapeDtypeStruct((VH, 2 * D), jnp.float32),
    )(table, table)


def _pool_body(idx_hbm, off_hbm, table_hbm, out_hbm, idx_v, off_v,
               rows0, rows1, out_v, sem0, sem1):
    wid = lax.axis_index("s") * NC + lax.axis_index("c")
    pltpu.sync_copy(idx_hbm.at[pl.ds(wid * IPW, IPW)], idx_v)
    pltpu.sync_copy(off_hbm.at[pl.ds(wid * IPW, IPW)], off_v)
    pltpu.async_copy(table_hbm.at[idx_v.at[pl.ds(0, IPC)]], rows0, sem0)
    pltpu.async_copy(table_hbm.at[idx_v.at[pl.ds(IPC, IPC)]], rows1, sem1)

    def outer(g, carry):
        for s, (rows, sem) in enumerate(((rows0, sem0), (rows1, sem1))):
            j = 2 * g + s
            pltpu.make_async_copy(table_hbm.at[pl.ds(0, IPC)], rows, sem).wait()
            ovecs = [off_v[pl.ds(j * IPC + k * 16, 16)] for k in range(IPC // 16)]
            for bl in range(BPC):
                r0 = bl * H
                o = ovecs[r0 // 16][r0 % 16]
                acc = [rows[r0, pl.ds(o + c * 16, 16)] for c in range(DV)]
                for l in range(1, H):
                    r = bl * H + l
                    o = ovecs[r // 16][r % 16]
                    for c in range(DV):
                        acc[c] = acc[c] + rows[r, pl.ds(o + c * 16, 16)]
                row_out = j * BPC + bl
                for c in range(DV):
                    out_v[row_out, pl.ds(c * 16, 16)] = acc[c] * (1.0 / H)

            @pl.when(j + 2 < NCH)
            def _():
                pltpu.async_copy(
                    table_hbm.at[idx_v.at[pl.ds((j + 2) * IPC, IPC)]], rows, sem)
        return carry

    lax.fori_loop(0, NCH // 2, outer, 0)
    pltpu.sync_copy(out_v, out_hbm.at[pl.ds(wid * BPW, BPW)])


def _pool(idx, off, packed):
    f = pl.kernel(
        _pool_body,
        out_type=jax.ShapeDtypeStruct((B, D), jnp.float32),
        mesh=plsc.VectorSubcoreMesh(core_axis_name="c", subcore_axis_name="s",
                                    num_cores=NC, num_subcores=NS),
        scratch_types=[
            pltpu.VMEM((IPW,), jnp.int32),
            pltpu.VMEM((IPW,), jnp.int32),
            pltpu.VMEM((IPC, 2 * D), jnp.float32),
            pltpu.VMEM((IPC, 2 * D), jnp.float32),
            pltpu.VMEM((BPW, D), jnp.float32),
            pltpu.SemaphoreType.DMA,
            pltpu.SemaphoreType.DMA,
        ],
    )
    return f(idx, off, packed)


def _mlp_body(pooled_ref, state_ref, swt_ref, sb_ref, w1t_ref, b1_ref,
              w2t_ref, b2_ref, out_ref):
    x = pooled_ref[...] + jnp.dot(state_ref[...], swt_ref[...],
                                  preferred_element_type=jnp.float32)
    x = x + sb_ref[...]
    h = jnp.maximum(x, 0.0)
    h = jnp.dot(h, w1t_ref[...], preferred_element_type=jnp.float32)
    h = jnp.maximum(h + b1_ref[...], 0.0)
    out_ref[...] = jnp.dot(h, w2t_ref[...],
                           preferred_element_type=jnp.float32) + b2_ref[...]


def _mlp(pooled, state, swt, sb, w1t, b1, w2t, b2):
    blk = 2048
    grid = B // blk
    rep = lambda shape: pl.BlockSpec(shape, lambda i: (0, 0))
    return pl.pallas_call(
        _mlp_body,
        grid=(grid,),
        in_specs=[
            pl.BlockSpec((blk, D), lambda i: (i, 0)),
            pl.BlockSpec((blk, NUM_OUT), lambda i: (i, 0)),
            rep((NUM_OUT, D)),
            rep((1, D)),
            rep((D, D // 2)),
            rep((1, D // 2)),
            rep((D // 2, NUM_OUT)),
            rep((1, NUM_OUT)),
        ],
        out_specs=pl.BlockSpec((blk, NUM_OUT), lambda i: (i, 0)),
        out_shape=jax.ShapeDtypeStruct((B, NUM_OUT), jnp.float32),
    )(pooled, state, swt, sb, w1t, b1, w2t, b2)


def kernel(players, state, emb_table, state_W, state_b, W1, b1, W2, b2):
    pi = players.astype(jnp.int32)
    q = (pi % VH).reshape(-1)
    off = jnp.where(pi >= VH, 64, 0).astype(jnp.int32).reshape(-1)
    packed = _repack(emb_table)
    pooled = _pool(q, off, packed)
    return _mlp(pooled, state,
                state_W.T, state_b.reshape(1, D),
                W1.T, b1.reshape(1, D // 2),
                W2.T, b2.reshape(1, NUM_OUT))


# speedup vs baseline: 1.2365x; 1.2365x over previous
import functools

import jax
import jax.numpy as jnp
from jax import lax
from jax.experimental import pallas as pl
from jax.experimental.pallas import tpu as pltpu
from jax.experimental.pallas import tpu_sc as plsc

B = 16384
H = 20
D = 64
NUM_OUT = 3
V = 1000000
VH = V // 2

NC = 2
NS = 16
NW = NC * NS
BPW = B // NW
BPC = 4
IPC = BPC * H
NCH = BPW // BPC
IPW = BPW * H
DV = D // 16


CB = 2048
CBH = CB // 2
NB = (V + CB - 1) // CB
VP = NB * CBH


def _repack_body(in_ref, out_ref):
    t = in_ref[...].T
    out_ref[:, 0:D] = t[0:CBH, :]
    out_ref[:, D:2 * D] = t[CBH:CB, :]


def _repack(table_t):
    return pl.pallas_call(
        _repack_body,
        grid=(NB,),
        in_specs=[pl.BlockSpec((D, CB), lambda i: (0, i))],
        out_specs=pl.BlockSpec((CBH, 2 * D), lambda i: (i, 0)),
        out_shape=jax.ShapeDtypeStruct((VP, 2 * D), jnp.float32),
    )(table_t)


def _pool_body(idx_hbm, off_hbm, table_hbm, out_hbm, idx_v, off_v,
               rows0, rows1, out_v, sem0, sem1):
    wid = lax.axis_index("s") * NC + lax.axis_index("c")
    pltpu.sync_copy(idx_hbm.at[pl.ds(wid * IPW, IPW)], idx_v)
    pltpu.sync_copy(off_hbm.at[pl.ds(wid * IPW, IPW)], off_v)
    pltpu.async_copy(table_hbm.at[idx_v.at[pl.ds(0, IPC)]], rows0, sem0)
    pltpu.async_copy(table_hbm.at[idx_v.at[pl.ds(IPC, IPC)]], rows1, sem1)

    def outer(g, carry):
        for s, (rows, sem) in enumerate(((rows0, sem0), (rows1, sem1))):
            j = 2 * g + s
            pltpu.make_async_copy(table_hbm.at[pl.ds(0, IPC)], rows, sem).wait()
            ovecs = [off_v[pl.ds(j * IPC + k * 16, 16)] for k in range(IPC // 16)]
            for bl in range(BPC):
                r0 = bl * H
                o = ovecs[r0 // 16][r0 % 16]
                acc = [rows[r0, pl.ds(o + c * 16, 16)] for c in range(DV)]
                for l in range(1, H):
                    r = bl * H + l
                    o = ovecs[r // 16][r % 16]
                    for c in range(DV):
                        acc[c] = acc[c] + rows[r, pl.ds(o + c * 16, 16)]
                row_out = j * BPC + bl
                for c in range(DV):
                    out_v[row_out, pl.ds(c * 16, 16)] = acc[c] * (1.0 / H)

            @pl.when(j + 2 < NCH)
            def _():
                pltpu.async_copy(
                    table_hbm.at[idx_v.at[pl.ds((j + 2) * IPC, IPC)]], rows, sem)
        return carry

    lax.fori_loop(0, NCH // 2, outer, 0)
    pltpu.sync_copy(out_v, out_hbm.at[pl.ds(wid * BPW, BPW)])


def _pool(idx, off, packed):
    f = pl.kernel(
        _pool_body,
        out_type=jax.ShapeDtypeStruct((B, D), jnp.float32),
        mesh=plsc.VectorSubcoreMesh(core_axis_name="c", subcore_axis_name="s",
                                    num_cores=NC, num_subcores=NS),
        scratch_types=[
            pltpu.VMEM((IPW,), jnp.int32),
            pltpu.VMEM((IPW,), jnp.int32),
            pltpu.VMEM((IPC, 2 * D), jnp.float32),
            pltpu.VMEM((IPC, 2 * D), jnp.float32),
            pltpu.VMEM((BPW, D), jnp.float32),
            pltpu.SemaphoreType.DMA,
            pltpu.SemaphoreType.DMA,
        ],
    )
    return f(idx, off, packed)


def _mlp_body(pooled_ref, state_ref, swt_ref, sb_ref, w1t_ref, b1_ref,
              w2t_ref, b2_ref, out_ref):
    x = pooled_ref[...] + jnp.dot(state_ref[...], swt_ref[...],
                                  preferred_element_type=jnp.float32)
    x = x + sb_ref[...]
    h = jnp.maximum(x, 0.0)
    h = jnp.dot(h, w1t_ref[...], preferred_element_type=jnp.float32)
    h = jnp.maximum(h + b1_ref[...], 0.0)
    out_ref[...] = jnp.dot(h, w2t_ref[...],
                           preferred_element_type=jnp.float32) + b2_ref[...]


def _mlp(pooled, state, swt, sb, w1t, b1, w2t, b2):
    blk = 2048
    grid = B // blk
    rep = lambda shape: pl.BlockSpec(shape, lambda i: (0, 0))
    return pl.pallas_call(
        _mlp_body,
        grid=(grid,),
        in_specs=[
            pl.BlockSpec((blk, D), lambda i: (i, 0)),
            pl.BlockSpec((blk, NUM_OUT), lambda i: (i, 0)),
            rep((NUM_OUT, D)),
            rep((1, D)),
            rep((D, D // 2)),
            rep((1, D // 2)),
            rep((D // 2, NUM_OUT)),
            rep((1, NUM_OUT)),
        ],
        out_specs=pl.BlockSpec((blk, NUM_OUT), lambda i: (i, 0)),
        out_shape=jax.ShapeDtypeStruct((B, NUM_OUT), jnp.float32),
    )(pooled, state, swt, sb, w1t, b1, w2t, b2)


def kernel(players, state, emb_table, state_W, state_b, W1, b1, W2, b2):
    pi = players.astype(jnp.int32)
    blk, w = pi // CB, pi % CB
    q = (blk * CBH + w % CBH).reshape(-1)
    off = ((w // CBH) * D).astype(jnp.int32).reshape(-1)
    packed = _repack(emb_table.T)
    pooled = _pool(q, off, packed)
    return _mlp(pooled, state,
                state_W.T, state_b.reshape(1, D),
                W1.T, b1.reshape(1, D // 2),
                W2.T, b2.reshape(1, NUM_OUT))


# speedup vs baseline: 1.5318x; 1.2388x over previous
import functools

import jax
import jax.numpy as jnp
from jax import lax
from jax.experimental import pallas as pl
from jax.experimental.pallas import tpu as pltpu
from jax.experimental.pallas import tpu_sc as plsc

B = 16384
H = 20
D = 64
NUM_OUT = 3
V = 1000000
VH = V // 2

NC = 2
NS = 16
NW = NC * NS
BPW = B // NW
BPC = 4
IPC = BPC * H
NCH = BPW // BPC
IPW = BPW * H
DV = D // 16


CB = 4096
CBH = CB // 2
NB = (V + CB - 1) // CB
VP = NB * CBH


def _repack_body(in_ref, out_ref):
    t1 = in_ref[:, 0:CBH][...].T
    t2 = in_ref[:, CBH:CB][...].T
    out_ref[...] = jnp.concatenate([t1, t2], axis=1)


def _repack(table_t):
    return pl.pallas_call(
        _repack_body,
        grid=(NB,),
        in_specs=[pl.BlockSpec((D, CB), lambda i: (0, i))],
        out_specs=pl.BlockSpec((CBH, 2 * D), lambda i: (i, 0)),
        out_shape=jax.ShapeDtypeStruct((VP, 2 * D), jnp.float32),
    )(table_t)


def _pool_body(idx_hbm, off_hbm, table_hbm, out_hbm, idx_v, off_v,
               rows0, rows1, out_v, sem0, sem1):
    wid = lax.axis_index("s") * NC + lax.axis_index("c")
    pltpu.sync_copy(idx_hbm.at[pl.ds(wid * IPW, IPW)], idx_v)
    pltpu.sync_copy(off_hbm.at[pl.ds(wid * IPW, IPW)], off_v)
    pltpu.async_copy(table_hbm.at[idx_v.at[pl.ds(0, IPC)]], rows0, sem0)
    pltpu.async_copy(table_hbm.at[idx_v.at[pl.ds(IPC, IPC)]], rows1, sem1)

    def outer(g, carry):
        for s, (rows, sem) in enumerate(((rows0, sem0), (rows1, sem1))):
            j = 2 * g + s
            pltpu.make_async_copy(table_hbm.at[pl.ds(0, IPC)], rows, sem).wait()
            ovecs = [off_v[pl.ds(j * IPC + k * 16, 16)] for k in range(IPC // 16)]
            for bl in range(BPC):
                r0 = bl * H
                o = ovecs[r0 // 16][r0 % 16]
                acc = [rows[r0, pl.ds(o + c * 16, 16)] for c in range(DV)]
                for l in range(1, H):
                    r = bl * H + l
                    o = ovecs[r // 16][r % 16]
                    for c in range(DV):
                        acc[c] = acc[c] + rows[r, pl.ds(o + c * 16, 16)]
                row_out = j * BPC + bl
                for c in range(DV):
                    out_v[row_out, pl.ds(c * 16, 16)] = acc[c] * (1.0 / H)

            @pl.when(j + 2 < NCH)
            def _():
                pltpu.async_copy(
                    table_hbm.at[idx_v.at[pl.ds((j + 2) * IPC, IPC)]], rows, sem)
        return carry

    lax.fori_loop(0, NCH // 2, outer, 0)
    pltpu.sync_copy(out_v, out_hbm.at[pl.ds(wid * BPW, BPW)])


def _pool(idx, off, packed):
    f = pl.kernel(
        _pool_body,
        out_type=jax.ShapeDtypeStruct((B, D), jnp.float32),
        mesh=plsc.VectorSubcoreMesh(core_axis_name="c", subcore_axis_name="s",
                                    num_cores=NC, num_subcores=NS),
        scratch_types=[
            pltpu.VMEM((IPW,), jnp.int32),
            pltpu.VMEM((IPW,), jnp.int32),
            pltpu.VMEM((IPC, 2 * D), jnp.float32),
            pltpu.VMEM((IPC, 2 * D), jnp.float32),
            pltpu.VMEM((BPW, D), jnp.float32),
            pltpu.SemaphoreType.DMA,
            pltpu.SemaphoreType.DMA,
        ],
    )
    return f(idx, off, packed)


def _mlp_body(pooled_ref, state_ref, swt_ref, sb_ref, w1t_ref, b1_ref,
              w2t_ref, b2_ref, out_ref):
    x = pooled_ref[...] + jnp.dot(state_ref[...], swt_ref[...],
                                  preferred_element_type=jnp.float32)
    x = x + sb_ref[...]
    h = jnp.maximum(x, 0.0)
    h = jnp.dot(h, w1t_ref[...], preferred_element_type=jnp.float32)
    h = jnp.maximum(h + b1_ref[...], 0.0)
    out_ref[...] = jnp.dot(h, w2t_ref[...],
                           preferred_element_type=jnp.float32) + b2_ref[...]


def _mlp(pooled, state, swt, sb, w1t, b1, w2t, b2):
    blk = 2048
    grid = B // blk
    rep = lambda shape: pl.BlockSpec(shape, lambda i: (0, 0))
    return pl.pallas_call(
        _mlp_body,
        grid=(grid,),
        in_specs=[
            pl.BlockSpec((blk, D), lambda i: (i, 0)),
            pl.BlockSpec((blk, NUM_OUT), lambda i: (i, 0)),
            rep((NUM_OUT, D)),
            rep((1, D)),
            rep((D, D // 2)),
            rep((1, D // 2)),
            rep((D // 2, NUM_OUT)),
            rep((1, NUM_OUT)),
        ],
        out_specs=pl.BlockSpec((blk, NUM_OUT), lambda i: (i, 0)),
        out_shape=jax.ShapeDtypeStruct((B, NUM_OUT), jnp.float32),
    )(pooled, state, swt, sb, w1t, b1, w2t, b2)


def kernel(players, state, emb_table, state_W, state_b, W1, b1, W2, b2):
    pi = players.astype(jnp.int32)
    blk, w = pi // CB, pi % CB
    q = (blk * CBH + w % CBH).reshape(-1)
    off = ((w // CBH) * D).astype(jnp.int32).reshape(-1)
    packed = _repack(emb_table.T)
    pooled = _pool(q, off, packed)
    return _mlp(pooled, state,
                state_W.T, state_b.reshape(1, D),
                W1.T, b1.reshape(1, D // 2),
                W2.T, b2.reshape(1, NUM_OUT))


# speedup vs baseline: 2.2237x; 1.4517x over previous
import functools

import jax
import jax.numpy as jnp
from jax import lax
from jax.experimental import pallas as pl
from jax.experimental.pallas import tpu as pltpu
from jax.experimental.pallas import tpu_sc as plsc

B = 16384
H = 20
D = 64
NUM_OUT = 3
V = 1000000
VH = V // 2

NC = 2
NS = 16
NW = NC * NS
BPW = B // NW
BPC = 4
IPC = BPC * H
NCH = BPW // BPC
IPW = BPW * H
DV = D // 16


CB = 8192
CBH = CB // 2
NB = (V + CB - 1) // CB
VP = NB * CBH


def _repack_body(in_ref, out_ref):
    x = in_ref[...]
    s = jnp.concatenate([x[:, 0:CBH], x[:, CBH:CB]], axis=0)
    out_ref[...] = s.T


def _repack(table_t):
    return pl.pallas_call(
        _repack_body,
        grid=(NB,),
        in_specs=[pl.BlockSpec((D, CB), lambda i: (0, i))],
        out_specs=pl.BlockSpec((CBH, 2 * D), lambda i: (i, 0)),
        out_shape=jax.ShapeDtypeStruct((VP, 2 * D), jnp.float32),
    )(table_t)


NSLOT = 4


def _pool_body(idx_hbm, off_hbm, table_hbm, out_hbm, idx_v, off_v,
               rows0, rows1, rows2, rows3, out_v, sem0, sem1, sem2, sem3):
    wid = lax.axis_index("s") * NC + lax.axis_index("c")
    slots = ((rows0, sem0), (rows1, sem1), (rows2, sem2), (rows3, sem3))
    pltpu.sync_copy(idx_hbm.at[pl.ds(wid * IPW, IPW)], idx_v)
    pltpu.sync_copy(off_hbm.at[pl.ds(wid * IPW, IPW)], off_v)
    for s, (rows, sem) in enumerate(slots):
        pltpu.async_copy(table_hbm.at[idx_v.at[pl.ds(s * IPC, IPC)]], rows, sem)

    def outer(g, carry):
        for s, (rows, sem) in enumerate(slots):
            j = NSLOT * g + s
            pltpu.make_async_copy(table_hbm.at[pl.ds(0, IPC)], rows, sem).wait()
            ovecs = [off_v[pl.ds(j * IPC + k * 16, 16)] for k in range(IPC // 16)]
            for bl in range(BPC):
                r0 = bl * H
                o = ovecs[r0 // 16][r0 % 16]
                acc = [rows[r0, pl.ds(o + c * 16, 16)] for c in range(DV)]
                for l in range(1, H):
                    r = bl * H + l
                    o = ovecs[r // 16][r % 16]
                    for c in range(DV):
                        acc[c] = acc[c] + rows[r, pl.ds(o + c * 16, 16)]
                base = (j * BPC + bl) * D
                for c in range(DV):
                    out_v[pl.ds(base + c * 16, 16)] = acc[c] * (1.0 / H)

            @pl.when(j + NSLOT < NCH)
            def _():
                pltpu.async_copy(
                    table_hbm.at[idx_v.at[pl.ds((j + NSLOT) * IPC, IPC)]], rows, sem)
        return carry

    lax.fori_loop(0, NCH // NSLOT, outer, 0)
    pltpu.sync_copy(out_v, out_hbm.at[pl.ds(wid * BPW * D, BPW * D)])


def _pool(idx, off, packed):
    f = pl.kernel(
        _pool_body,
        out_type=jax.ShapeDtypeStruct((B * D,), jnp.float32),
        mesh=plsc.VectorSubcoreMesh(core_axis_name="c", subcore_axis_name="s",
                                    num_cores=NC, num_subcores=NS),
        scratch_types=[
            pltpu.VMEM((IPW,), jnp.int32),
            pltpu.VMEM((IPW,), jnp.int32),
            pltpu.VMEM((IPC, 2 * D), jnp.float32),
            pltpu.VMEM((IPC, 2 * D), jnp.float32),
            pltpu.VMEM((IPC, 2 * D), jnp.float32),
            pltpu.VMEM((IPC, 2 * D), jnp.float32),
            pltpu.VMEM((BPW * D,), jnp.float32),
            pltpu.SemaphoreType.DMA,
            pltpu.SemaphoreType.DMA,
            pltpu.SemaphoreType.DMA,
            pltpu.SemaphoreType.DMA,
        ],
    )
    return f(idx, off, packed)


def _mlp_body(pooled_ref, state_ref, swt_ref, sb_ref, w1t_ref, b1_ref,
              w2t_ref, b2_ref, out_ref):
    x = pooled_ref[...] + jnp.dot(state_ref[...], swt_ref[...],
                                  preferred_element_type=jnp.float32)
    x = x + sb_ref[...]
    h = jnp.maximum(x, 0.0)
    h = jnp.dot(h, w1t_ref[...], preferred_element_type=jnp.float32)
    h = jnp.maximum(h + b1_ref[...], 0.0)
    out_ref[...] = jnp.dot(h, w2t_ref[...],
                           preferred_element_type=jnp.float32) + b2_ref[...]


def _mlp(pooled, state, swt, sb, w1t, b1, w2t, b2):
    blk = 2048
    grid = B // blk
    rep = lambda shape: pl.BlockSpec(shape, lambda i: (0, 0))
    return pl.pallas_call(
        _mlp_body,
        grid=(grid,),
        in_specs=[
            pl.BlockSpec((blk, D), lambda i: (i, 0)),
            pl.BlockSpec((blk, NUM_OUT), lambda i: (i, 0)),
            rep((NUM_OUT, D)),
            rep((1, D)),
            rep((D, D // 2)),
            rep((1, D // 2)),
            rep((D // 2, NUM_OUT)),
            rep((1, NUM_OUT)),
        ],
        out_specs=pl.BlockSpec((blk, NUM_OUT), lambda i: (i, 0)),
        out_shape=jax.ShapeDtypeStruct((B, NUM_OUT), jnp.float32),
    )(pooled, state, swt, sb, w1t, b1, w2t, b2)


def kernel(players, state, emb_table, state_W, state_b, W1, b1, W2, b2):
    pi = players.astype(jnp.int32)
    blk, w = pi // CB, pi % CB
    q = (blk * CBH + w % CBH).reshape(-1)
    off = ((w // CBH) * D).astype(jnp.int32).reshape(-1)
    packed = _repack(emb_table.T)
    pooled = _pool(q, off, packed).reshape(B, D)
    return _mlp(pooled, state,
                state_W.T, state_b.reshape(1, D),
                W1.T, b1.reshape(1, D // 2),
                W2.T, b2.reshape(1, NUM_OUT))
